# Initial kernel scaffold; baseline (speedup 1.0000x reference)
import jax
import jax.numpy as jnp
from jax.experimental import pallas as pl


def kernel(pd):
    raise NotImplementedError("write your pallas kernel here")


# speedup vs baseline: 106.1871x; 106.1871x over previous
import functools

import jax
import jax.numpy as jnp
from jax import lax
from jax.experimental import pallas as pl
from jax.experimental.pallas import tpu as pltpu
from jax.experimental.pallas import tpu_sc as plsc

T = 100
TPAD = 112
NV = TPAD // 16
KTOP = 2
B, D, C, P = 32, 2, 3, 2048
NEG = -2.0


def _pl_sc_call(pd_flat):
    mesh = plsc.VectorSubcoreMesh(core_axis_name="c", subcore_axis_name="s")

    @functools.partial(
        pl.kernel,
        mesh=mesh,
        out_type=jax.ShapeDtypeStruct((B, D, KTOP, C, TPAD), jnp.float32),
        scratch_types=[
            pltpu.VMEM((C * P * 2,), jnp.float32),
            pltpu.VMEM((KTOP, C, TPAD), jnp.float32),
        ],
    )
    def sc_kernel(pd_hbm, out_hbm, in_v, out_v):
        wid = lax.axis_index("s") * 2 + lax.axis_index("c")
        lane = lax.iota(jnp.int32, 16).astype(jnp.float32)
        tvecs = [(lane + float(16 * j)) * (1.0 / (T - 1)) for j in range(NV)]

        def update(m1, m2, bb, dd):
            nm1, nm2 = [], []
            for j in range(NV):
                v = jnp.minimum(tvecs[j] - bb, dd - tvecs[j])
                nm2.append(jnp.maximum(m2[j], jnp.minimum(m1[j], v)))
                nm1.append(jnp.maximum(m1[j], v))
            return tuple(nm1), tuple(nm2)

        for d in range(D):
            pltpu.sync_copy(pd_hbm.at[wid, d], in_v)
            nbars = P - 1 if d == 0 else P
            nfull = nbars // 8
            ntail = nbars % 8

            for c in range(C):
                base = c * P * 2

                def body(k, carry, base=base):
                    m1, m2 = carry
                    w = in_v[pl.ds(base + 16 * k, 16)]
                    for i in range(8):
                        m1, m2 = update(m1, m2, w[2 * i], w[2 * i + 1])
                    return m1, m2

                init = (
                    tuple(jnp.full((16,), NEG, jnp.float32) for _ in range(NV)),
                    tuple(jnp.full((16,), NEG, jnp.float32) for _ in range(NV)),
                )
                m1, m2 = lax.fori_loop(0, nfull, body, init)
                if ntail:
                    w = in_v[pl.ds(base + 16 * nfull, 16)]
                    for i in range(ntail):
                        m1, m2 = update(m1, m2, w[2 * i], w[2 * i + 1])

                zero = jnp.zeros((16,), jnp.float32)
                for j in range(NV):
                    out_v[0, c, pl.ds(16 * j, 16)] = jnp.maximum(m1[j], zero)
                    out_v[1, c, pl.ds(16 * j, 16)] = jnp.maximum(m2[j], zero)

            pltpu.sync_copy(out_v, out_hbm.at[wid, d])

    return sc_kernel(pd_flat)


@jax.jit
def kernel(pd):
    pd_flat = pd.reshape(B, D, C * P * 2)
    out = _pl_sc_call(pd_flat)
    return out[..., :T].reshape(B, D, KTOP, C * T)
